# Initial kernel scaffold; baseline (speedup 1.0000x reference)
import jax
import jax.numpy as jnp
from jax.experimental import pallas as pl


def kernel(x, edge_index, Wp, bp, W1, b1, W2, b2, gamma, beta):
    raise NotImplementedError("write your pallas kernel here")


# speedup vs baseline: 1.5447x; 1.5447x over previous
import functools

import jax
import jax.numpy as jnp
from jax import lax
from jax.experimental import pallas as pl
from jax.experimental.pallas import tpu as pltpu
from jax.experimental.pallas import tpu_sc as plsc

N = 10000
D = 256
E = 160000

NW = 32
RNG = 313
RNG_LAST = N - (NW - 1) * RNG

C_SCAN = 1600
F_FLUSH = 1024
S_STAGE = 1280
E_PAD = E + S_STAGE
G_AGG = 128

_VMESH = plsc.VectorSubcoreMesh(core_axis_name="c", subcore_axis_name="s")
_SC_PARAMS = pltpu.CompilerParams(needs_layout_passes=False)


def _worker_id():
    return lax.axis_index("s") * 2 + lax.axis_index("c")


def _partition_body(src_hbm, dst_hbm, srcl_hbm, dstl_hbm, cnt_hbm,
                    srcc, dstc, sstage, dstage, cnt_vmem):
    w = _worker_id()
    lo = w * RNG
    hi = jnp.minimum(lo + RNG, N)
    base = pl.multiple_of(w * E_PAD, 8)

    zero16 = jnp.zeros((16,), jnp.int32)

    @pl.loop(0, S_STAGE, step=16)
    def _(i):
        sstage[pl.ds(i, 16)] = zero16
        dstage[pl.ds(i, 16)] = zero16

    def scan_chunk(ci, carry):
        c, total = carry
        off = ci * C_SCAN
        pltpu.sync_copy(src_hbm.at[pl.ds(off, C_SCAN)], srcc)
        pltpu.sync_copy(dst_hbm.at[pl.ds(off, C_SCAN)], dstc)

        def vec_step(k, carry2):
            c, total = carry2
            vd = dstc[pl.ds(k * 16, 16)]
            vs = srcc[pl.ds(k * 16, 16)]
            m = (vd >= lo) & (vd < hi)
            plsc.store_compressed(sstage.at[pl.ds(c, 16)], vs, mask=m)
            plsc.store_compressed(dstage.at[pl.ds(c, 16)], vd - lo, mask=m)
            nadd = jnp.max(plsc.all_reduce_population_count(m))
            c = c + nadd
            flush = c >= F_FLUSH

            @pl.when(flush)
            def _():
                o = pl.multiple_of(base + total, 8)
                pltpu.sync_copy(sstage, srcl_hbm.at[pl.ds(o, S_STAGE)])
                pltpu.sync_copy(dstage, dstl_hbm.at[pl.ds(o, S_STAGE)])
                sstage[pl.ds(0, 16)] = sstage[pl.ds(F_FLUSH, 16)]
                dstage[pl.ds(0, 16)] = dstage[pl.ds(F_FLUSH, 16)]

            shift = jnp.where(flush, F_FLUSH, 0)
            return c - shift, total + shift

        return lax.fori_loop(0, C_SCAN // 16, vec_step, (c, total))

    c, total = lax.fori_loop(0, E // C_SCAN, scan_chunk,
                             (jnp.int32(0), jnp.int32(0)))
    o = pl.multiple_of(base + total, 8)
    pltpu.sync_copy(sstage, srcl_hbm.at[pl.ds(o, S_STAGE)])
    pltpu.sync_copy(dstage, dstl_hbm.at[pl.ds(o, S_STAGE)])
    cnt_vmem[...] = jnp.full((16,), total + c, jnp.int32)
    pltpu.sync_copy(cnt_vmem, cnt_hbm.at[pl.ds(pl.multiple_of(w * 16, 8), 16)])


def _sc_partition(src, dst):
    i32 = jnp.int32
    out_types = (jax.ShapeDtypeStruct((NW * E_PAD,), i32),
                 jax.ShapeDtypeStruct((NW * E_PAD,), i32),
                 jax.ShapeDtypeStruct((NW * 16,), i32))
    scratch = [pltpu.VMEM((C_SCAN,), i32),
               pltpu.VMEM((C_SCAN,), i32),
               pltpu.VMEM((S_STAGE,), i32),
               pltpu.VMEM((S_STAGE,), i32),
               pltpu.VMEM((16,), i32)]
    return pl.kernel(_partition_body, out_type=out_types, mesh=_VMESH,
                     compiler_params=_SC_PARAMS,
                     scratch_types=scratch)(src, dst)


def _agg_body(h_hbm, srcl_hbm, dstl_hbm, cnt_hbm, agg_hbm,
              idx_v, rows_v, acc, dl_vmem, cnt_vmem):
    w = _worker_id()
    lo = w * RNG
    base = pl.multiple_of(w * E_PAD, 8)

    pltpu.sync_copy(cnt_hbm.at[pl.ds(pl.multiple_of(w * 16, 8), 16)], cnt_vmem)
    n = cnt_vmem[...][0]

    neg = jnp.full((16,), -jnp.inf, jnp.float32)

    @pl.loop(0, (RNG + 1) * D, step=16)
    def _(i):
        acc[pl.ds(i, 16)] = neg

    num_chunks = (n + (G_AGG - 1)) // G_AGG

    def chunk(gi, _):
        off = gi * G_AGG
        o = pl.multiple_of(base + off, 8)
        pltpu.sync_copy(srcl_hbm.at[pl.ds(o, G_AGG)], idx_v)
        pltpu.sync_copy(dstl_hbm.at[pl.ds(o, G_AGG)], dl_vmem.at[pl.ds(0, G_AGG)])
        pltpu.sync_copy(h_hbm.at[idx_v], rows_v)

        def edge(e, _):
            b = dl_vmem[pl.ds(e, 16)][0] * D
            b = jnp.where(off + e < n, b, RNG * D)
            for j in range(D // 16):
                sl = pl.ds(b + j * 16, 16)
                acc[sl] = jnp.maximum(acc[sl], rows_v[e, pl.ds(j * 16, 16)])
            return 0

        lax.fori_loop(0, G_AGG, edge, 0)
        return 0

    lax.fori_loop(0, num_chunks, chunk, 0)

    od = pl.multiple_of(lo * D, 8)

    @pl.when(w < NW - 1)
    def _():
        pltpu.sync_copy(acc.at[pl.ds(0, RNG * D)],
                        agg_hbm.at[pl.ds(od, RNG * D)])

    @pl.when(w == NW - 1)
    def _():
        pltpu.sync_copy(acc.at[pl.ds(0, RNG_LAST * D)],
                        agg_hbm.at[pl.ds(od, RNG_LAST * D)])


def _sc_aggregate(h, srcl, dstl, cnt):
    scratch = [pltpu.VMEM((G_AGG,), jnp.int32),
               pltpu.VMEM((G_AGG, D), jnp.float32),
               pltpu.VMEM(((RNG + 1) * D,), jnp.float32),
               pltpu.VMEM((G_AGG + 16,), jnp.int32),
               pltpu.VMEM((16,), jnp.int32)]
    out = pl.kernel(_agg_body,
                    out_type=jax.ShapeDtypeStruct((N * D,), jnp.float32),
                    mesh=_VMESH, compiler_params=_SC_PARAMS,
                    scratch_types=scratch)(h, srcl, dstl, cnt)
    return out.reshape(N, D)


R_BLK = 1000


def _gate_body(x_ref, wp_ref, bp_ref, o_ref):
    x = x_ref[...]
    t = jnp.dot(x, wp_ref[...], preferred_element_type=jnp.float32)
    o_ref[...] = x * jax.nn.sigmoid(t + bp_ref[...])


def _tc_gate(x, Wp, bp):
    return pl.pallas_call(
        _gate_body,
        grid=(N // R_BLK,),
        in_specs=[pl.BlockSpec((R_BLK, D), lambda i: (i, 0)),
                  pl.BlockSpec((D, D), lambda i: (0, 0)),
                  pl.BlockSpec((1, D), lambda i: (0, 0))],
        out_specs=pl.BlockSpec((R_BLK, D), lambda i: (i, 0)),
        out_shape=jax.ShapeDtypeStruct((N, D), jnp.float32),
    )(x, Wp, bp.reshape(1, D))


def _mlp_body(h_ref, a_ref, w1_ref, b1_ref, w2_ref, b2_ref, z2_ref, s_ref):
    a = a_ref[...]
    a = jnp.where(jnp.isneginf(a), 0.0, a)
    z = h_ref[...] + a
    z1 = jnp.maximum(jnp.dot(z, w1_ref[...],
                             preferred_element_type=jnp.float32) + b1_ref[...], 0.0)
    z2 = jnp.maximum(jnp.dot(z1, w2_ref[...],
                             preferred_element_type=jnp.float32) + b2_ref[...], 0.0)
    z2_ref[...] = z2

    @pl.when(pl.program_id(0) == 0)
    def _():
        s_ref[...] = jnp.zeros_like(s_ref)

    s1 = jnp.sum(z2, axis=0, keepdims=True)
    s2 = jnp.sum(z2 * z2, axis=0, keepdims=True)
    s_ref[...] += jnp.concatenate(
        [s1, s2, jnp.zeros((6, D), jnp.float32)], axis=0)


def _tc_mlp(h, agg, W1i, b1i, W2i, b2i):
    return pl.pallas_call(
        _mlp_body,
        grid=(N // R_BLK,),
        in_specs=[pl.BlockSpec((R_BLK, D), lambda i: (i, 0)),
                  pl.BlockSpec((R_BLK, D), lambda i: (i, 0)),
                  pl.BlockSpec((D, D), lambda i: (0, 0)),
                  pl.BlockSpec((1, D), lambda i: (0, 0)),
                  pl.BlockSpec((D, D), lambda i: (0, 0)),
                  pl.BlockSpec((1, D), lambda i: (0, 0))],
        out_specs=[pl.BlockSpec((R_BLK, D), lambda i: (i, 0)),
                   pl.BlockSpec((8, D), lambda i: (0, 0))],
        out_shape=[jax.ShapeDtypeStruct((N, D), jnp.float32),
                   jax.ShapeDtypeStruct((8, D), jnp.float32)],
    )(h, agg, W1i, b1i.reshape(1, D), W2i, b2i.reshape(1, D))


def _bn_body(z_ref, sc_ref, sh_ref, o_ref):
    o_ref[...] = jnp.tanh(z_ref[...] * sc_ref[...] + sh_ref[...])


def _tc_bn(z2, scale, shift):
    return pl.pallas_call(
        _bn_body,
        grid=(N // R_BLK,),
        in_specs=[pl.BlockSpec((R_BLK, D), lambda i: (i, 0)),
                  pl.BlockSpec((1, D), lambda i: (0, 0)),
                  pl.BlockSpec((1, D), lambda i: (0, 0))],
        out_specs=pl.BlockSpec((R_BLK, D), lambda i: (i, 0)),
        out_shape=jax.ShapeDtypeStruct((N, D), jnp.float32),
    )(z2, scale.reshape(1, D), shift.reshape(1, D))


@jax.jit
def _run(x, edge_index, Wp, bp, W1, b1, W2, b2, gamma, beta):
    src = edge_index[0]
    dst = edge_index[1]
    h = _tc_gate(x, Wp, bp)
    srcl, dstl, cnt = _sc_partition(src, dst)
    outs = [h]
    for i in range(3):
        agg = _sc_aggregate(h, srcl, dstl, cnt)
        z2, sums = _tc_mlp(h, agg, W1[i], b1[i], W2[i], b2[i])
        mu = sums[0] / N
        var = sums[1] / N - mu * mu
        scale = gamma[i] / jnp.sqrt(var + 1e-5)
        shift = beta[i] - mu * scale
        h = _tc_bn(z2, scale, shift)
        outs.append(h)
    return tuple(outs)


def kernel(x, edge_index, Wp, bp, W1, b1, W2, b2, gamma, beta):
    return _run(x, edge_index, Wp, bp, W1, b1, W2, b2, gamma, beta)


# speedup vs baseline: 2.1168x; 1.3703x over previous
import jax
import jax.numpy as jnp
from jax import lax
from jax.experimental import pallas as pl
from jax.experimental.pallas import tpu as pltpu
from jax.experimental.pallas import tpu_sc as plsc

N = 10000
D = 256
E = 160000

NW = 32
RNG = 313
RNG_LAST = N - (NW - 1) * RNG

C_SCAN = 1600
F_FLUSH = 1024
S_STAGE = 1280
E_PAD = E + S_STAGE
G_AGG = 64
SRC_BITS = 14
SRC_MASK = (1 << SRC_BITS) - 1

_VMESH = plsc.VectorSubcoreMesh(core_axis_name="c", subcore_axis_name="s")
_SC_PARAMS = pltpu.CompilerParams(needs_layout_passes=False)


def _worker_id():
    return lax.axis_index("s") * 2 + lax.axis_index("c")


def _partition_body(src_hbm, dst_hbm, list_hbm, cnt_hbm,
                    srcc, dstc, srcc2, dstc2, pstage, cnt_vmem,
                    ssem0, ssem1):
    w = _worker_id()
    lo = w * RNG
    hi = jnp.minimum(lo + RNG, N)
    base = pl.multiple_of(w * E_PAD, 8)

    zero16 = jnp.zeros((16,), jnp.int32)

    @pl.loop(0, S_STAGE, step=16)
    def _(i):
        pstage[pl.ds(i, 16)] = zero16

    def fire_chunk(ci, sbuf, dbuf, sem):
        pltpu.async_copy(src_hbm.at[pl.ds(ci * C_SCAN, C_SCAN)], sbuf, sem)
        pltpu.async_copy(dst_hbm.at[pl.ds(ci * C_SCAN, C_SCAN)], dbuf, sem)

    def scan_chunk(ci, carry, sbuf, dbuf, sem):
        c, total = carry
        pltpu.make_async_copy(src_hbm.at[pl.ds(0, C_SCAN)], sbuf, sem).wait()
        pltpu.make_async_copy(src_hbm.at[pl.ds(0, C_SCAN)], dbuf, sem).wait()

        def vec_step(k, carry2):
            c, total = carry2
            vd = dbuf[pl.ds(k * 16, 16)]
            vs = sbuf[pl.ds(k * 16, 16)]
            m = (vd >= lo) & (vd < hi)
            packed = vs | ((vd - lo) << SRC_BITS)
            plsc.store_compressed(pstage.at[pl.ds(c, 16)], packed, mask=m)
            nadd = jnp.max(plsc.all_reduce_population_count(m))
            c = c + nadd
            flush = c >= F_FLUSH

            @pl.when(flush)
            def _():
                o = pl.multiple_of(base + total, 8)
                pltpu.sync_copy(pstage, list_hbm.at[pl.ds(o, S_STAGE)])
                pstage[pl.ds(0, 16)] = pstage[pl.ds(F_FLUSH, 16)]

            shift = jnp.where(flush, F_FLUSH, 0)
            return c - shift, total + shift

        return lax.fori_loop(0, C_SCAN // 16, vec_step, (c, total))

    NCH = E // C_SCAN
    fire_chunk(0, srcc, dstc, ssem0)
    fire_chunk(1, srcc2, dstc2, ssem1)

    def scan_pair(p, carry):
        carry = scan_chunk(2 * p, carry, srcc, dstc, ssem0)

        @pl.when(2 * p + 2 < NCH)
        def _():
            fire_chunk(2 * p + 2, srcc, dstc, ssem0)

        carry = scan_chunk(2 * p + 1, carry, srcc2, dstc2, ssem1)

        @pl.when(2 * p + 3 < NCH)
        def _():
            fire_chunk(2 * p + 3, srcc2, dstc2, ssem1)

        return carry

    c, total = lax.fori_loop(0, NCH // 2, scan_pair,
                             (jnp.int32(0), jnp.int32(0)))
    o = pl.multiple_of(base + total, 8)
    pltpu.sync_copy(pstage, list_hbm.at[pl.ds(o, S_STAGE)])
    cnt_vmem[...] = jnp.full((16,), total + c, jnp.int32)
    pltpu.sync_copy(cnt_vmem, cnt_hbm.at[pl.ds(pl.multiple_of(w * 16, 8), 16)])


def _sc_partition(src, dst):
    i32 = jnp.int32
    out_types = (jax.ShapeDtypeStruct((NW * E_PAD,), i32),
                 jax.ShapeDtypeStruct((NW * 16,), i32))
    scratch = [pltpu.VMEM((C_SCAN,), i32),
               pltpu.VMEM((C_SCAN,), i32),
               pltpu.VMEM((C_SCAN,), i32),
               pltpu.VMEM((C_SCAN,), i32),
               pltpu.VMEM((S_STAGE,), i32),
               pltpu.VMEM((16,), i32),
               pltpu.SemaphoreType.DMA, pltpu.SemaphoreType.DMA]
    return pl.kernel(_partition_body, out_type=out_types, mesh=_VMESH,
                     compiler_params=_SC_PARAMS,
                     scratch_types=scratch)(src, dst)


def _agg_body(h_hbm, list_hbm, cnt_hbm, agg_hbm,
              lbuf0, lbuf1, ibuf0, ibuf1, rows0, rows1, dlb, acc, cnt_vmem,
              lsem0, lsem1, gsem0, gsem1):
    w = _worker_id()
    lo = w * RNG
    base = pl.multiple_of(w * E_PAD, 8)

    pltpu.sync_copy(cnt_hbm.at[pl.ds(pl.multiple_of(w * 16, 8), 16)], cnt_vmem)
    n = cnt_vmem[...][0]

    neg = jnp.full((16,), -jnp.inf, jnp.float32)

    @pl.loop(0, (RNG + 1) * D, step=16)
    def _(i):
        acc[pl.ds(i, 16)] = neg

    nc = (n + (G_AGG - 1)) // G_AGG
    iota16 = lax.iota(jnp.int32, 16)

    def list_slice(j):
        o = pl.multiple_of(base + j * G_AGG, 8)
        return list_hbm.at[pl.ds(o, G_AGG)]

    def build_idx(lb, ib):
        @pl.loop(0, G_AGG, step=16)
        def _(t):
            ib[pl.ds(t, 16)] = lb[pl.ds(t, 16)] & SRC_MASK

    @pl.when(nc > 0)
    def _():
        pltpu.sync_copy(list_slice(0), lbuf0)
        build_idx(lbuf0, ibuf0)
        pltpu.async_copy(h_hbm.at[ibuf0], rows0, gsem0)

    @pl.when(nc > 1)
    def _():
        pltpu.async_copy(list_slice(1), lbuf1, lsem1)

    def step(g, lb, ib, rb, o_lb, o_ib, o_rb, lsem_k, lsem_o, gsem_k, gsem_o):
        @pl.when(g + 1 < nc)
        def _():
            pltpu.make_async_copy(list_slice(0), o_lb, lsem_o).wait()
            build_idx(o_lb, o_ib)
            pltpu.async_copy(h_hbm.at[o_ib], o_rb, gsem_o)

        goff = g * G_AGG

        @pl.loop(0, G_AGG, step=16)
        def _(t):
            v = lb[pl.ds(t, 16)] >> SRC_BITS
            valid = (goff + t + iota16) < n
            dlb[pl.ds(t, 16)] = jnp.where(valid, v, RNG)

        @pl.when(g + 2 < nc)
        def _():
            pltpu.async_copy(list_slice(g + 2), lb, lsem_k)

        pltpu.make_async_copy(h_hbm.at[ib], rb, gsem_k).wait()

        @pl.loop(0, G_AGG // 16)
        def _(q):
            dlv = dlb[pl.ds(q * 16, 16)]
            for i in range(16):
                b = dlv[i] * D
                e = q * 16 + i
                for j in range(D // 16):
                    sl = pl.ds(b + j * 16, 16)
                    acc[sl] = jnp.maximum(acc[sl], rb[e, pl.ds(j * 16, 16)])

    def pair(p, _):
        g = p * 2

        @pl.when(g < nc)
        def _():
            step(g, lbuf0, ibuf0, rows0, lbuf1, ibuf1, rows1,
                 lsem0, lsem1, gsem0, gsem1)

        @pl.when(g + 1 < nc)
        def _():
            step(g + 1, lbuf1, ibuf1, rows1, lbuf0, ibuf0, rows0,
                 lsem1, lsem0, gsem1, gsem0)

        return 0

    lax.fori_loop(0, (nc + 1) // 2, pair, 0)

    od = pl.multiple_of(lo * D, 8)

    @pl.when(w < NW - 1)
    def _():
        pltpu.sync_copy(acc.at[pl.ds(0, RNG * D)],
                        agg_hbm.at[pl.ds(od, RNG * D)])

    @pl.when(w == NW - 1)
    def _():
        pltpu.sync_copy(acc.at[pl.ds(0, RNG_LAST * D)],
                        agg_hbm.at[pl.ds(od, RNG_LAST * D)])


def _sc_aggregate(h, plist, cnt):
    i32 = jnp.int32
    f32 = jnp.float32
    scratch = [pltpu.VMEM((G_AGG,), i32), pltpu.VMEM((G_AGG,), i32),
               pltpu.VMEM((G_AGG,), i32), pltpu.VMEM((G_AGG,), i32),
               pltpu.VMEM((G_AGG, D), f32), pltpu.VMEM((G_AGG, D), f32),
               pltpu.VMEM((G_AGG,), i32),
               pltpu.VMEM(((RNG + 1) * D,), f32),
               pltpu.VMEM((16,), i32),
               pltpu.SemaphoreType.DMA, pltpu.SemaphoreType.DMA,
               pltpu.SemaphoreType.DMA, pltpu.SemaphoreType.DMA]
    out = pl.kernel(_agg_body,
                    out_type=jax.ShapeDtypeStruct((N * D,), f32),
                    mesh=_VMESH, compiler_params=_SC_PARAMS,
                    scratch_types=scratch)(h, plist, cnt)
    return out.reshape(N, D)


R_BLK = 1000


def _gate_body(x_ref, wp_ref, bp_ref, o_ref):
    x = x_ref[...]
    t = jnp.dot(x, wp_ref[...], preferred_element_type=jnp.float32)
    o_ref[...] = x * jax.nn.sigmoid(t + bp_ref[...])


def _tc_gate(x, Wp, bp):
    return pl.pallas_call(
        _gate_body,
        grid=(N // R_BLK,),
        in_specs=[pl.BlockSpec((R_BLK, D), lambda i: (i, 0)),
                  pl.BlockSpec((D, D), lambda i: (0, 0)),
                  pl.BlockSpec((1, D), lambda i: (0, 0))],
        out_specs=pl.BlockSpec((R_BLK, D), lambda i: (i, 0)),
        out_shape=jax.ShapeDtypeStruct((N, D), jnp.float32),
    )(x, Wp, bp.reshape(1, D))


def _mlp_body(h_ref, a_ref, w1_ref, b1_ref, w2_ref, b2_ref, z2_ref, s_ref):
    a = a_ref[...]
    a = jnp.where(jnp.isneginf(a), 0.0, a)
    z = h_ref[...] + a
    z1 = jnp.maximum(jnp.dot(z, w1_ref[...],
                             preferred_element_type=jnp.float32) + b1_ref[...], 0.0)
    z2 = jnp.maximum(jnp.dot(z1, w2_ref[...],
                             preferred_element_type=jnp.float32) + b2_ref[...], 0.0)
    z2_ref[...] = z2

    @pl.when(pl.program_id(0) == 0)
    def _():
        s_ref[...] = jnp.zeros_like(s_ref)

    s1 = jnp.sum(z2, axis=0, keepdims=True)
    s2 = jnp.sum(z2 * z2, axis=0, keepdims=True)
    s_ref[...] += jnp.concatenate(
        [s1, s2, jnp.zeros((6, D), jnp.float32)], axis=0)


def _tc_mlp(h, agg, W1i, b1i, W2i, b2i):
    return pl.pallas_call(
        _mlp_body,
        grid=(N // R_BLK,),
        in_specs=[pl.BlockSpec((R_BLK, D), lambda i: (i, 0)),
                  pl.BlockSpec((R_BLK, D), lambda i: (i, 0)),
                  pl.BlockSpec((D, D), lambda i: (0, 0)),
                  pl.BlockSpec((1, D), lambda i: (0, 0)),
                  pl.BlockSpec((D, D), lambda i: (0, 0)),
                  pl.BlockSpec((1, D), lambda i: (0, 0))],
        out_specs=[pl.BlockSpec((R_BLK, D), lambda i: (i, 0)),
                   pl.BlockSpec((8, D), lambda i: (0, 0))],
        out_shape=[jax.ShapeDtypeStruct((N, D), jnp.float32),
                   jax.ShapeDtypeStruct((8, D), jnp.float32)],
    )(h, agg, W1i, b1i.reshape(1, D), W2i, b2i.reshape(1, D))


def _bn_body(z_ref, sc_ref, sh_ref, o_ref):
    o_ref[...] = jnp.tanh(z_ref[...] * sc_ref[...] + sh_ref[...])


def _tc_bn(z2, scale, shift):
    return pl.pallas_call(
        _bn_body,
        grid=(N // R_BLK,),
        in_specs=[pl.BlockSpec((R_BLK, D), lambda i: (i, 0)),
                  pl.BlockSpec((1, D), lambda i: (0, 0)),
                  pl.BlockSpec((1, D), lambda i: (0, 0))],
        out_specs=pl.BlockSpec((R_BLK, D), lambda i: (i, 0)),
        out_shape=jax.ShapeDtypeStruct((N, D), jnp.float32),
    )(z2, scale.reshape(1, D), shift.reshape(1, D))


@jax.jit
def _run(x, edge_index, Wp, bp, W1, b1, W2, b2, gamma, beta):
    src = edge_index[0]
    dst = edge_index[1]
    h = _tc_gate(x, Wp, bp)
    plist, cnt = _sc_partition(src, dst)
    outs = [h]
    for i in range(3):
        agg = _sc_aggregate(h, plist, cnt)
        z2, sums = _tc_mlp(h, agg, W1[i], b1[i], W2[i], b2[i])
        mu = sums[0] / N
        var = sums[1] / N - mu * mu
        scale = gamma[i] / jnp.sqrt(var + 1e-5)
        shift = beta[i] - mu * scale
        h = _tc_bn(z2, scale, shift)
        outs.append(h)
    return tuple(outs)


def kernel(x, edge_index, Wp, bp, W1, b1, W2, b2, gamma, beta):
    return _run(x, edge_index, Wp, bp, W1, b1, W2, b2, gamma, beta)


# speedup vs baseline: 2.2047x; 1.0416x over previous
import jax
import jax.numpy as jnp
from jax import lax
from jax.experimental import pallas as pl
from jax.experimental.pallas import tpu as pltpu
from jax.experimental.pallas import tpu_sc as plsc

N = 10000
D = 256
E = 160000

NW = 32
RNG = 313
RNG_LAST = N - (NW - 1) * RNG

C_SCAN = 1600
F_FLUSH = 1024
S_STAGE = 1280
E_PAD = E + S_STAGE
G_AGG = 80
SRC_BITS = 14
SRC_MASK = (1 << SRC_BITS) - 1

_VMESH = plsc.VectorSubcoreMesh(core_axis_name="c", subcore_axis_name="s")
_SC_PARAMS = pltpu.CompilerParams(needs_layout_passes=False)


def _worker_id():
    return lax.axis_index("s") * 2 + lax.axis_index("c")


def _partition_body(src_hbm, dst_hbm, list_hbm, cnt_hbm,
                    srcc, dstc, srcc2, dstc2, pstage, cnt_vmem,
                    ssem0, ssem1):
    w = _worker_id()
    lo = w * RNG
    hi = jnp.minimum(lo + RNG, N)
    base = pl.multiple_of(w * E_PAD, 8)

    zero16 = jnp.zeros((16,), jnp.int32)

    @pl.loop(0, S_STAGE, step=16)
    def _(i):
        pstage[pl.ds(i, 16)] = zero16

    def fire_chunk(ci, sbuf, dbuf, sem):
        pltpu.async_copy(src_hbm.at[pl.ds(ci * C_SCAN, C_SCAN)], sbuf, sem)
        pltpu.async_copy(dst_hbm.at[pl.ds(ci * C_SCAN, C_SCAN)], dbuf, sem)

    def scan_chunk(ci, carry, sbuf, dbuf, sem):
        c, total = carry
        pltpu.make_async_copy(src_hbm.at[pl.ds(0, C_SCAN)], sbuf, sem).wait()
        pltpu.make_async_copy(src_hbm.at[pl.ds(0, C_SCAN)], dbuf, sem).wait()

        def vec_step(k, carry2):
            c, total = carry2
            vd = dbuf[pl.ds(k * 16, 16)]
            vs = sbuf[pl.ds(k * 16, 16)]
            m = (vd >= lo) & (vd < hi)
            packed = vs | ((vd - lo) << SRC_BITS)
            plsc.store_compressed(pstage.at[pl.ds(c, 16)], packed, mask=m)
            nadd = plsc.all_reduce_population_count(m)[0]
            c = c + nadd
            flush = c >= F_FLUSH

            @pl.when(flush)
            def _():
                o = pl.multiple_of(base + total, 8)
                pltpu.sync_copy(pstage, list_hbm.at[pl.ds(o, S_STAGE)])
                pstage[pl.ds(0, 16)] = pstage[pl.ds(F_FLUSH, 16)]

            shift = jnp.where(flush, F_FLUSH, 0)
            return c - shift, total + shift

        return lax.fori_loop(0, C_SCAN // 16, vec_step, (c, total))

    NCH = E // C_SCAN
    fire_chunk(0, srcc, dstc, ssem0)
    fire_chunk(1, srcc2, dstc2, ssem1)

    def scan_pair(p, carry):
        carry = scan_chunk(2 * p, carry, srcc, dstc, ssem0)

        @pl.when(2 * p + 2 < NCH)
        def _():
            fire_chunk(2 * p + 2, srcc, dstc, ssem0)

        carry = scan_chunk(2 * p + 1, carry, srcc2, dstc2, ssem1)

        @pl.when(2 * p + 3 < NCH)
        def _():
            fire_chunk(2 * p + 3, srcc2, dstc2, ssem1)

        return carry

    c, total = lax.fori_loop(0, NCH // 2, scan_pair,
                             (jnp.int32(0), jnp.int32(0)))
    o = pl.multiple_of(base + total, 8)
    pltpu.sync_copy(pstage, list_hbm.at[pl.ds(o, S_STAGE)])
    cnt_vmem[...] = jnp.full((16,), total + c, jnp.int32)
    pltpu.sync_copy(cnt_vmem, cnt_hbm.at[pl.ds(pl.multiple_of(w * 16, 8), 16)])


def _sc_partition(src, dst):
    i32 = jnp.int32
    out_types = (jax.ShapeDtypeStruct((NW * E_PAD,), i32),
                 jax.ShapeDtypeStruct((NW * 16,), i32))
    scratch = [pltpu.VMEM((C_SCAN,), i32),
               pltpu.VMEM((C_SCAN,), i32),
               pltpu.VMEM((C_SCAN,), i32),
               pltpu.VMEM((C_SCAN,), i32),
               pltpu.VMEM((S_STAGE,), i32),
               pltpu.VMEM((16,), i32),
               pltpu.SemaphoreType.DMA, pltpu.SemaphoreType.DMA]
    return pl.kernel(_partition_body, out_type=out_types, mesh=_VMESH,
                     compiler_params=_SC_PARAMS,
                     scratch_types=scratch)(src, dst)


def _agg_body(h_hbm, list_hbm, cnt_hbm, agg_hbm,
              lbuf0, lbuf1, ibuf0, ibuf1, rows0, rows1, dlb, acc, cnt_vmem,
              lsem0, lsem1, gsem0, gsem1):
    w = _worker_id()
    lo = w * RNG
    base = pl.multiple_of(w * E_PAD, 8)

    pltpu.sync_copy(cnt_hbm.at[pl.ds(pl.multiple_of(w * 16, 8), 16)], cnt_vmem)
    n = cnt_vmem[...][0]

    neg = jnp.full((16,), -jnp.inf, jnp.float32)

    @pl.loop(0, (RNG + 1) * D, step=16)
    def _(i):
        acc[pl.ds(i, 16)] = neg

    nc = (n + (G_AGG - 1)) // G_AGG
    iota16 = lax.iota(jnp.int32, 16)

    def list_slice(j):
        o = pl.multiple_of(base + j * G_AGG, 8)
        return list_hbm.at[pl.ds(o, G_AGG)]

    def build_idx(lb, ib):
        @pl.loop(0, G_AGG, step=16)
        def _(t):
            ib[pl.ds(t, 16)] = lb[pl.ds(t, 16)] & SRC_MASK

    @pl.when(nc > 0)
    def _():
        pltpu.sync_copy(list_slice(0), lbuf0)
        build_idx(lbuf0, ibuf0)
        pltpu.async_copy(h_hbm.at[ibuf0], rows0, gsem0)

    @pl.when(nc > 1)
    def _():
        pltpu.async_copy(list_slice(1), lbuf1, lsem1)

    def step(g, lb, ib, rb, o_lb, o_ib, o_rb, lsem_k, lsem_o, gsem_k, gsem_o):
        @pl.when(g + 1 < nc)
        def _():
            pltpu.make_async_copy(list_slice(0), o_lb, lsem_o).wait()
            build_idx(o_lb, o_ib)
            pltpu.async_copy(h_hbm.at[o_ib], o_rb, gsem_o)

        goff = g * G_AGG

        @pl.loop(0, G_AGG, step=16)
        def _(t):
            v = lb[pl.ds(t, 16)] >> SRC_BITS
            valid = (goff + t + iota16) < n
            dlb[pl.ds(t, 16)] = jnp.where(valid, v, RNG)

        @pl.when(g + 2 < nc)
        def _():
            pltpu.async_copy(list_slice(g + 2), lb, lsem_k)

        pltpu.make_async_copy(h_hbm.at[ib], rb, gsem_k).wait()

        @pl.loop(0, G_AGG // 16)
        def _(q):
            dlv = dlb[pl.ds(q * 16, 16)]
            for i in range(16):
                b = dlv[i] * D
                e = q * 16 + i
                for j in range(D // 16):
                    sl = pl.ds(b + j * 16, 16)
                    acc[sl] = jnp.maximum(acc[sl], rb[e, pl.ds(j * 16, 16)])

    def pair(p, _):
        g = p * 2

        @pl.when(g < nc)
        def _():
            step(g, lbuf0, ibuf0, rows0, lbuf1, ibuf1, rows1,
                 lsem0, lsem1, gsem0, gsem1)

        @pl.when(g + 1 < nc)
        def _():
            step(g + 1, lbuf1, ibuf1, rows1, lbuf0, ibuf0, rows0,
                 lsem1, lsem0, gsem1, gsem0)

        return 0

    lax.fori_loop(0, (nc + 1) // 2, pair, 0)

    od = pl.multiple_of(lo * D, 8)

    @pl.when(w < NW - 1)
    def _():
        pltpu.sync_copy(acc.at[pl.ds(0, RNG * D)],
                        agg_hbm.at[pl.ds(od, RNG * D)])

    @pl.when(w == NW - 1)
    def _():
        pltpu.sync_copy(acc.at[pl.ds(0, RNG_LAST * D)],
                        agg_hbm.at[pl.ds(od, RNG_LAST * D)])


def _sc_aggregate(h, plist, cnt):
    i32 = jnp.int32
    f32 = jnp.float32
    scratch = [pltpu.VMEM((G_AGG,), i32), pltpu.VMEM((G_AGG,), i32),
               pltpu.VMEM((G_AGG,), i32), pltpu.VMEM((G_AGG,), i32),
               pltpu.VMEM((G_AGG, D), f32), pltpu.VMEM((G_AGG, D), f32),
               pltpu.VMEM((G_AGG,), i32),
               pltpu.VMEM(((RNG + 1) * D,), f32),
               pltpu.VMEM((16,), i32),
               pltpu.SemaphoreType.DMA, pltpu.SemaphoreType.DMA,
               pltpu.SemaphoreType.DMA, pltpu.SemaphoreType.DMA]
    out = pl.kernel(_agg_body,
                    out_type=jax.ShapeDtypeStruct((N * D,), f32),
                    mesh=_VMESH, compiler_params=_SC_PARAMS,
                    scratch_types=scratch)(h, plist, cnt)
    return out.reshape(N, D)


R_BLK = 2000


def _gate_body(x_ref, wp_ref, bp_ref, o_ref):
    x = x_ref[...]
    t = jnp.dot(x, wp_ref[...], preferred_element_type=jnp.float32)
    o_ref[...] = x * jax.nn.sigmoid(t + bp_ref[...])


def _tc_gate(x, Wp, bp):
    return pl.pallas_call(
        _gate_body,
        grid=(N // R_BLK,),
        in_specs=[pl.BlockSpec((R_BLK, D), lambda i: (i, 0)),
                  pl.BlockSpec((D, D), lambda i: (0, 0)),
                  pl.BlockSpec((1, D), lambda i: (0, 0))],
        out_specs=pl.BlockSpec((R_BLK, D), lambda i: (i, 0)),
        out_shape=jax.ShapeDtypeStruct((N, D), jnp.float32),
    )(x, Wp, bp.reshape(1, D))


def _mlp_body(h_ref, a_ref, w1_ref, b1_ref, w2_ref, b2_ref, z2_ref, s_ref):
    a = a_ref[...]
    a = jnp.where(jnp.isneginf(a), 0.0, a)
    z = h_ref[...] + a
    z1 = jnp.maximum(jnp.dot(z, w1_ref[...],
                             preferred_element_type=jnp.float32) + b1_ref[...], 0.0)
    z2 = jnp.maximum(jnp.dot(z1, w2_ref[...],
                             preferred_element_type=jnp.float32) + b2_ref[...], 0.0)
    z2_ref[...] = z2

    @pl.when(pl.program_id(0) == 0)
    def _():
        s_ref[...] = jnp.zeros_like(s_ref)

    s1 = jnp.sum(z2, axis=0, keepdims=True)
    s2 = jnp.sum(z2 * z2, axis=0, keepdims=True)
    s_ref[...] += jnp.concatenate(
        [s1, s2, jnp.zeros((6, D), jnp.float32)], axis=0)


def _tc_mlp(h, agg, W1i, b1i, W2i, b2i):
    return pl.pallas_call(
        _mlp_body,
        grid=(N // R_BLK,),
        in_specs=[pl.BlockSpec((R_BLK, D), lambda i: (i, 0)),
                  pl.BlockSpec((R_BLK, D), lambda i: (i, 0)),
                  pl.BlockSpec((D, D), lambda i: (0, 0)),
                  pl.BlockSpec((1, D), lambda i: (0, 0)),
                  pl.BlockSpec((D, D), lambda i: (0, 0)),
                  pl.BlockSpec((1, D), lambda i: (0, 0))],
        out_specs=[pl.BlockSpec((R_BLK, D), lambda i: (i, 0)),
                   pl.BlockSpec((8, D), lambda i: (0, 0))],
        out_shape=[jax.ShapeDtypeStruct((N, D), jnp.float32),
                   jax.ShapeDtypeStruct((8, D), jnp.float32)],
    )(h, agg, W1i, b1i.reshape(1, D), W2i, b2i.reshape(1, D))


def _bn_body(z_ref, sc_ref, sh_ref, o_ref):
    o_ref[...] = jnp.tanh(z_ref[...] * sc_ref[...] + sh_ref[...])


def _tc_bn(z2, scale, shift):
    return pl.pallas_call(
        _bn_body,
        grid=(N // R_BLK,),
        in_specs=[pl.BlockSpec((R_BLK, D), lambda i: (i, 0)),
                  pl.BlockSpec((1, D), lambda i: (0, 0)),
                  pl.BlockSpec((1, D), lambda i: (0, 0))],
        out_specs=pl.BlockSpec((R_BLK, D), lambda i: (i, 0)),
        out_shape=jax.ShapeDtypeStruct((N, D), jnp.float32),
    )(z2, scale.reshape(1, D), shift.reshape(1, D))


@jax.jit
def _run(x, edge_index, Wp, bp, W1, b1, W2, b2, gamma, beta):
    src = edge_index[0]
    dst = edge_index[1]
    h = _tc_gate(x, Wp, bp)
    plist, cnt = _sc_partition(src, dst)
    outs = [h]
    for i in range(3):
        agg = _sc_aggregate(h, plist, cnt)
        z2, sums = _tc_mlp(h, agg, W1[i], b1[i], W2[i], b2[i])
        mu = sums[0] / N
        var = sums[1] / N - mu * mu
        scale = gamma[i] / jnp.sqrt(var + 1e-5)
        shift = beta[i] - mu * scale
        h = _tc_bn(z2, scale, shift)
        outs.append(h)
    return tuple(outs)


def kernel(x, edge_index, Wp, bp, W1, b1, W2, b2, gamma, beta):
    return _run(x, edge_index, Wp, bp, W1, b1, W2, b2, gamma, beta)
